# baseline (speedup 1.0000x reference)
import functools

import jax
import jax.numpy as jnp
from jax import lax
from jax.experimental import pallas as pl
from jax.experimental.pallas import tpu as pltpu
from jax.experimental.pallas import tpu_sc as plsc

N_NODES = 10000
N_AGENTS = 1000
N_EDGES = 320000
D = 128
H1 = 256
NC = 2
NS = 16
NW = NC * NS
NH = N_EDGES // 2
EPW = NH // NW
BLK = 512
CAP = 5120
RCH = 64
NBW = CAP // BLK
MAXB = NW * NBW


def _relu(v):
    return jnp.maximum(v, 0.0)


def _stage1_body(x_ref, wm1_ref, bm1_ref, xws_ref, xwd_ref):
    x = x_ref[...]
    xws_ref[...] = jnp.dot(x, wm1_ref[:D, :], preferred_element_type=jnp.float32)
    xwd_ref[...] = (
        jnp.dot(x_ref[:N_AGENTS, :], wm1_ref[D:, :], preferred_element_type=jnp.float32)
        + bm1_ref[...]
    )


def _stage1(x, wm1, bm1r):
    return pl.pallas_call(
        _stage1_body,
        out_shape=[
            jax.ShapeDtypeStruct((N_NODES, H1), jnp.float32),
            jax.ShapeDtypeStruct((N_AGENTS, H1), jnp.float32),
        ],
    )(x, wm1, bm1r)


def _make_sc_body(e0):
    def _sc_body(ei_hbm, xws_hbm, xwd_hbm,
                 pres_hbm, pred_hbm, dstc_hbm, cnt_hbm,
                 src_v, dst_v, csrc_v, cdst_v, bufa, bufb, bufa2, bufb2,
                 cvec, sema, semb, semoa, semob):
        wid = lax.axis_index("s") * NC + lax.axis_index("c")
        ebase = e0 + wid * EPW
        pltpu.sync_copy(ei_hbm.at[pl.ds(ebase, EPW)], src_v.at[pl.ds(0, EPW)])
        pltpu.sync_copy(ei_hbm.at[pl.ds(N_EDGES + ebase, EPW)], dst_v.at[pl.ds(0, EPW)])

        one16 = jnp.ones((16,), jnp.int32)
        zer16 = jnp.zeros((16,), jnp.int32)
        lane = lax.iota(jnp.int32, 16)

        def cbody(i, cnt):
            b0 = i * 64
            cur = cnt
            for g in range(4):
                s = src_v[pl.ds(b0 + g * 16, 16)]
                dv = dst_v[pl.ds(b0 + g * 16, 16)]
                m = dv < N_AGENTS
                csum = plsc.cumsum(jnp.where(m, one16, zer16))
                pos = cur + csum - 1
                plsc.store_scatter(csrc_v, [pos], s, mask=m)
                plsc.store_scatter(cdst_v, [pos], dv, mask=m)
                cur = cur + csum[15]
            return cur

        cnt = lax.fori_loop(0, EPW // 64, cbody, jnp.int32(0))
        for b0 in range(EPW - EPW % 64, EPW, 16):
            nv = min(16, EPW - b0)
            s = src_v[pl.ds(b0, 16)]
            dv = dst_v[pl.ds(b0, 16)]
            m = dv < N_AGENTS
            if nv < 16:
                m = m & (lane < nv)
            csum = plsc.cumsum(jnp.where(m, one16, zer16))
            pos = cnt + csum - 1
            plsc.store_scatter(csrc_v, [pos], s, mask=m)
            plsc.store_scatter(cdst_v, [pos], dv, mask=m)
            cnt = cnt + csum[15]

        for t in range(RCH // 16):
            csrc_v[pl.ds(cnt + t * 16, 16)] = zer16
            cdst_v[pl.ds(cnt + t * 16, 16)] = zer16

        cvec[...] = zer16 + cnt
        pltpu.sync_copy(cvec, cnt_hbm.at[wid])

        obase = wid * CAP
        ncg = (cnt + (RCH - 1)) // RCH

        def g_issue(c, ba, bb, semg):
            off = c * RCH
            pltpu.async_copy(xws_hbm.at[csrc_v.at[pl.ds(off, RCH)]], ba, semg)
            pltpu.async_copy(xwd_hbm.at[cdst_v.at[pl.ds(off, RCH)]], bb, semg)

        def g_wait(ba, bb, semg):
            pltpu.make_async_copy(xws_hbm.at[pl.ds(0, RCH)], ba, semg).wait()
            pltpu.make_async_copy(xws_hbm.at[pl.ds(0, RCH)], bb, semg).wait()

        def o_issue(c, ba, bb, semo):
            off = c * RCH
            pltpu.async_copy(ba, pres_hbm.at[pl.ds(obase + off, RCH)], semo)
            pltpu.async_copy(bb, pred_hbm.at[pl.ds(obase + off, RCH)], semo)
            pltpu.async_copy(
                cdst_v.at[pl.ds(off, RCH)],
                dstc_hbm.at[(obase + off) // BLK, 0, pl.ds((obase + off) % BLK, RCH)],
                semo,
            )

        def o_wait(ba, bb, semo):
            pltpu.make_async_copy(ba, pres_hbm.at[pl.ds(obase, RCH)], semo).wait()
            pltpu.make_async_copy(bb, pred_hbm.at[pl.ds(obase, RCH)], semo).wait()
            pltpu.make_async_copy(
                cdst_v.at[pl.ds(0, RCH)],
                dstc_hbm.at[obase // BLK, 0, pl.ds(0, RCH)],
                semo,
            ).wait()

        @pl.when(ncg > 0)
        def _():
            g_issue(jnp.int32(0), bufa, bufb, sema)

        @pl.when(ncg > 1)
        def _():
            g_issue(jnp.int32(1), bufa2, bufb2, semb)

        def pbody(p, carry):
            c0 = 2 * p
            g_wait(bufa, bufb, sema)
            o_issue(c0, bufa, bufb, semoa)

            @pl.when(c0 + 1 < ncg)
            def _():
                g_wait(bufa2, bufb2, semb)
                o_issue(c0 + 1, bufa2, bufb2, semob)

            @pl.when(c0 + 2 < ncg)
            def _():
                o_wait(bufa, bufb, semoa)
                g_issue(c0 + 2, bufa, bufb, sema)

            @pl.when(c0 + 3 < ncg)
            def _():
                o_wait(bufa2, bufb2, semob)
                g_issue(c0 + 3, bufa2, bufb2, semb)

            return carry

        lax.fori_loop(0, (ncg + 1) // 2, pbody, 0)

        last_even = lax.rem(ncg - 1, 2) == 0

        @pl.when((ncg >= 1) & last_even)
        def _():
            o_wait(bufa, bufb, semoa)

        @pl.when(ncg >= 2)
        def _():
            o_wait(bufa2, bufb2, semob)

            @pl.when(jnp.logical_not(last_even))
            def _():
                o_wait(bufa, bufb, semoa)

    return _sc_body


def _stage2(e0, edge_index_flat, xws, xwd):
    mesh = plsc.VectorSubcoreMesh(core_axis_name="c", subcore_axis_name="s")
    f = functools.partial(
        pl.kernel,
        mesh=mesh,
        out_type=[
            jax.ShapeDtypeStruct((NW * CAP, H1), jnp.float32),
            jax.ShapeDtypeStruct((NW * CAP, H1), jnp.float32),
            jax.ShapeDtypeStruct((MAXB, 1, BLK), jnp.int32),
            jax.ShapeDtypeStruct((NW, 16), jnp.int32),
        ],
        scratch_types=[
            pltpu.VMEM((EPW + 16, ), jnp.int32),
            pltpu.VMEM((EPW + 16, ), jnp.int32),
            pltpu.VMEM((CAP,), jnp.int32),
            pltpu.VMEM((CAP,), jnp.int32),
            pltpu.VMEM((RCH, H1), jnp.float32),
            pltpu.VMEM((RCH, H1), jnp.float32),
            pltpu.VMEM((RCH, H1), jnp.float32),
            pltpu.VMEM((RCH, H1), jnp.float32),
            pltpu.VMEM((16,), jnp.int32),
            pltpu.SemaphoreType.DMA,
            pltpu.SemaphoreType.DMA,
            pltpu.SemaphoreType.DMA,
            pltpu.SemaphoreType.DMA,
        ],
        compiler_params=pltpu.CompilerParams(needs_layout_passes=False),
    )(_make_sc_body(e0))
    return f(edge_index_flat, xws, xwd)


def _block_pass(cnt_s, pres_hbm, pred_hbm, dstc_hbm,
                wm2_ref, bm2_ref, wm3_ref, bm3_ref,
                sbuf, dbuf, dstv, aggr, rows_s, drow_s, vlim_s, sem):
    def touter(t, idx):
        cnt = cnt_s[t, 0]
        nb = (cnt + (BLK - 1)) // BLK

        def binner(b, idx2):
            rows_s[idx2] = t * CAP + b * BLK
            drow_s[idx2] = t * NBW + b
            vlim_s[idx2] = cnt - b * BLK
            return idx2 + 1

        return lax.fori_loop(0, nb, binner, idx)

    total = lax.fori_loop(0, NW, touter, jnp.int32(0))

    def issue(k):
        s = lax.rem(k, 2)
        row0 = pl.multiple_of(rows_s[k], BLK)
        dr = drow_s[k]
        pltpu.make_async_copy(pres_hbm.at[pl.ds(row0, BLK)], sbuf.at[s], sem.at[s, 0]).start()
        pltpu.make_async_copy(pred_hbm.at[pl.ds(row0, BLK)], dbuf.at[s], sem.at[s, 1]).start()
        pltpu.make_async_copy(dstc_hbm.at[dr], dstv.at[s], sem.at[s, 2]).start()

    def wait(k):
        s = lax.rem(k, 2)
        pltpu.make_async_copy(pres_hbm.at[pl.ds(0, BLK)], sbuf.at[s], sem.at[s, 0]).wait()
        pltpu.make_async_copy(pred_hbm.at[pl.ds(0, BLK)], dbuf.at[s], sem.at[s, 1]).wait()
        pltpu.make_async_copy(dstc_hbm.at[0], dstv.at[s], sem.at[s, 2]).wait()

    @pl.when(total > 0)
    def _():
        issue(jnp.int32(0))

    def kbody(k, carry):
        @pl.when(k + 1 < total)
        def _():
            issue(k + 1)

        wait(k)
        s = lax.rem(k, 2)
        vlim = vlim_s[k]
        rows = lax.broadcasted_iota(jnp.int32, (BLK, 1), 0)
        m1 = jnp.where(rows < vlim, _relu(sbuf[s] + dbuf[s]), 0.0)
        m2 = _relu(jnp.dot(m1, wm2_ref[...], preferred_element_type=jnp.float32)
                   + bm2_ref[...])
        m3 = (jnp.dot(m2, wm3_ref[...], preferred_element_type=jnp.float32)
              + bm3_ref[...])
        cols = lax.broadcasted_iota(jnp.int32, (1, BLK), 1)
        dsel = jnp.where(cols < vlim, dstv[s], N_AGENTS)
        oh = (lax.broadcasted_iota(jnp.int32, (N_AGENTS, BLK), 0) == dsel
              ).astype(jnp.float32)
        m3h = m3.astype(jnp.bfloat16).astype(jnp.float32)
        m3l = m3 - m3h
        aggr[...] = (aggr[...]
                     + jnp.dot(oh, m3h, preferred_element_type=jnp.float32)
                     + jnp.dot(oh, m3l, preferred_element_type=jnp.float32))
        return carry

    lax.fori_loop(0, total, kbody, 0)


def _critic_a_body(cnt_s, pres_hbm, pred_hbm, dstc_hbm,
                   wm2_ref, bm2_ref, wm3_ref, bm3_ref,
                   out_ref,
                   sbuf, dbuf, dstv, aggr, rows_s, drow_s, vlim_s, sem):
    aggr[...] = jnp.zeros_like(aggr)
    _block_pass(cnt_s, pres_hbm, pred_hbm, dstc_hbm,
                wm2_ref, bm2_ref, wm3_ref, bm3_ref,
                sbuf, dbuf, dstv, aggr, rows_s, drow_s, vlim_s, sem)
    out_ref[...] = aggr[...]


def _critic_b_body(cnt_s, pres_hbm, pred_hbm, dstc_hbm, aggr0_ref,
                   x_ref, act_ref,
                   wm2_ref, bm2_ref, wm3_ref, bm3_ref,
                   wa1_ref, ba1_ref, wa2_ref, ba2_ref,
                   wu1_ref, bu1_ref, wu2_ref, bu2_ref, wu3_ref, bu3_ref,
                   wact_ref, bact_ref, wh1_ref, bh1_ref, wh2_ref, bh2_ref,
                   wq_ref, bq_ref,
                   out_ref,
                   sbuf, dbuf, dstv, aggr, rows_s, drow_s, vlim_s, sem):
    aggr[...] = aggr0_ref[...]
    _block_pass(cnt_s, pres_hbm, pred_hbm, dstc_hbm,
                wm2_ref, bm2_ref, wm3_ref, bm3_ref,
                sbuf, dbuf, dstv, aggr, rows_s, drow_s, vlim_s, sem)

    ag = aggr[...]
    a = _relu(jnp.dot(ag, wa1_ref[...], preferred_element_type=jnp.float32) + ba1_ref[...])
    a = _relu(jnp.dot(a, wa2_ref[...], preferred_element_type=jnp.float32) + ba2_ref[...])
    h = _relu(jnp.dot(x_ref[...], wu1_ref[:D, :], preferred_element_type=jnp.float32)
              + jnp.dot(a, wu1_ref[D:, :], preferred_element_type=jnp.float32)
              + bu1_ref[...])
    h = _relu(jnp.dot(h, wu2_ref[...], preferred_element_type=jnp.float32) + bu2_ref[...])
    h = jnp.dot(h, wu3_ref[...], preferred_element_type=jnp.float32) + bu3_ref[...]
    ap = _relu(jnp.dot(act_ref[...], wact_ref[...], preferred_element_type=jnp.float32)
               + bact_ref[...])
    z = _relu(jnp.dot(h, wh1_ref[:D, :], preferred_element_type=jnp.float32)
              + jnp.dot(ap, wh1_ref[D:, :], preferred_element_type=jnp.float32)
              + bh1_ref[...])
    z = _relu(jnp.dot(z, wh2_ref[...], preferred_element_type=jnp.float32) + bh2_ref[...])
    q = jnp.sum(z * wq_ref[...], axis=1, keepdims=True) + bq_ref[...]
    out_ref[...] = q


_SCRATCH3 = [
    pltpu.VMEM((2, BLK, H1), jnp.float32),
    pltpu.VMEM((2, BLK, H1), jnp.float32),
    pltpu.VMEM((2, 1, BLK), jnp.int32),
    pltpu.VMEM((N_AGENTS, D), jnp.float32),
    pltpu.SMEM((MAXB,), jnp.int32),
    pltpu.SMEM((MAXB,), jnp.int32),
    pltpu.SMEM((MAXB,), jnp.int32),
    pltpu.SemaphoreType.DMA((2, 3)),
]


def _stage3a(counts, pres, pred, dstc, wm2, bm2r, wm3, bm3r):
    in_specs = [pl.BlockSpec(memory_space=pltpu.SMEM),
                pl.BlockSpec(memory_space=pl.ANY),
                pl.BlockSpec(memory_space=pl.ANY),
                pl.BlockSpec(memory_space=pl.ANY)]
    in_specs += [pl.BlockSpec(memory_space=pltpu.VMEM)] * 4
    return pl.pallas_call(
        _critic_a_body,
        out_shape=jax.ShapeDtypeStruct((N_AGENTS, D), jnp.float32),
        in_specs=in_specs,
        out_specs=pl.BlockSpec(memory_space=pltpu.VMEM),
        scratch_shapes=list(_SCRATCH3),
    )(counts, pres, pred, dstc, wm2, bm2r, wm3, bm3r)


def _stage3b(counts, pres, pred, dstc, aggr0, x_a, actions, weights):
    in_specs = [pl.BlockSpec(memory_space=pltpu.SMEM),
                pl.BlockSpec(memory_space=pl.ANY),
                pl.BlockSpec(memory_space=pl.ANY),
                pl.BlockSpec(memory_space=pl.ANY)]
    in_specs += [pl.BlockSpec(memory_space=pltpu.VMEM)] * (3 + len(weights))
    return pl.pallas_call(
        _critic_b_body,
        out_shape=jax.ShapeDtypeStruct((N_AGENTS, 1), jnp.float32),
        in_specs=in_specs,
        out_specs=pl.BlockSpec(memory_space=pltpu.VMEM),
        scratch_shapes=list(_SCRATCH3),
    )(counts, pres, pred, dstc, aggr0, x_a, actions, *weights)


def kernel(x, edge_index, actions,
           Wm1, bm1, Wm2, bm2, Wm3, bm3,
           Wa1, ba1, Wa2, ba2,
           Wu1, bu1, Wu2, bu2, Wu3, bu3,
           Wact, bact, Wh1, bh1, Wh2, bh2, Wq, bq):
    xws, xwd = _stage1(x, Wm1, bm1.reshape(1, -1))
    eif = edge_index.reshape(-1)
    pres_a, pred_a, dstc_a, counts_a = _stage2(0, eif, xws, xwd)
    pres_b, pred_b, dstc_b, counts_b = _stage2(NH, eif, xws, xwd)
    wm2, bm2r, wm3, bm3r = Wm2, bm2.reshape(1, -1), Wm3, bm3.reshape(1, -1)
    aggr0 = _stage3a(counts_a, pres_a, pred_a, dstc_a, wm2, bm2r, wm3, bm3r)
    weights = (wm2, bm2r, wm3, bm3r,
               Wa1, ba1.reshape(1, -1), Wa2, ba2.reshape(1, -1),
               Wu1, bu1.reshape(1, -1), Wu2, bu2.reshape(1, -1),
               Wu3, bu3.reshape(1, -1),
               Wact, bact.reshape(1, -1),
               Wh1, bh1.reshape(1, -1), Wh2, bh2.reshape(1, -1),
               Wq.reshape(1, -1), bq.reshape(1, 1))
    out = _stage3b(counts_b, pres_b, pred_b, dstc_b, aggr0,
                   x[:N_AGENTS], actions, weights)
    return out.reshape(N_AGENTS)


# speedup vs baseline: 1.2723x; 1.2723x over previous
import functools

import jax
import jax.numpy as jnp
from jax import lax
from jax.experimental import pallas as pl
from jax.experimental.pallas import tpu as pltpu
from jax.experimental.pallas import tpu_sc as plsc

N_NODES = 10000
N_AGENTS = 1000
N_EDGES = 320000
D = 128
H1 = 256
NC = 2
NS = 16
NW = NC * NS
EPW = N_EDGES // NW
BLK = 512
CAP = 10240
RCH = 64
NBW = CAP // BLK
MAXB = NW * NBW


def _relu(v):
    return jnp.maximum(v, 0.0)


def _stage1_body(x_ref, wm1_ref, bm1_ref, xws_ref, xwd_ref):
    x = x_ref[...]
    xws_ref[...] = jnp.dot(x, wm1_ref[:D, :], preferred_element_type=jnp.float32)
    xwd_ref[...] = (
        jnp.dot(x_ref[:N_AGENTS, :], wm1_ref[D:, :], preferred_element_type=jnp.float32)
        + bm1_ref[...]
    )


def _stage1(x, wm1, bm1r):
    return pl.pallas_call(
        _stage1_body,
        out_shape=[
            jax.ShapeDtypeStruct((N_NODES, H1), jnp.float32),
            jax.ShapeDtypeStruct((N_AGENTS, H1), jnp.float32),
        ],
    )(x, wm1, bm1r)


def _sc_body(ei_hbm, xws_hbm, xwd_hbm,
             pres_hbm, pred_hbm, dstc_hbm, cnt_hbm,
             src_v, dst_v, csrc_v, cdst_v, bufa, bufb, bufa2, bufb2, cvec, sema, semb, semoa, semob):
    wid = lax.axis_index("s") * NC + lax.axis_index("c")
    ebase = wid * EPW
    pltpu.sync_copy(ei_hbm.at[pl.ds(ebase, EPW)], src_v)
    pltpu.sync_copy(ei_hbm.at[pl.ds(N_EDGES + ebase, EPW)], dst_v)

    one16 = jnp.ones((16,), jnp.int32)
    zer16 = jnp.zeros((16,), jnp.int32)

    def cbody(i, cnt):
        b0 = i * 64
        cur = cnt
        for g in range(4):
            s = src_v[pl.ds(b0 + g * 16, 16)]
            dv = dst_v[pl.ds(b0 + g * 16, 16)]
            m = dv < N_AGENTS
            csum = plsc.cumsum(jnp.where(m, one16, zer16))
            pos = cur + csum - 1
            plsc.store_scatter(csrc_v, [pos], s, mask=m)
            plsc.store_scatter(cdst_v, [pos], dv, mask=m)
            cur = cur + csum[15]
        return cur

    cnt = lax.fori_loop(0, EPW // 64, cbody, jnp.int32(0))
    for b0 in range(EPW - EPW % 64, EPW, 16):
        s = src_v[pl.ds(b0, 16)]
        dv = dst_v[pl.ds(b0, 16)]
        m = dv < N_AGENTS
        csum = plsc.cumsum(jnp.where(m, one16, zer16))
        pos = cnt + csum - 1
        plsc.store_scatter(csrc_v, [pos], s, mask=m)
        plsc.store_scatter(cdst_v, [pos], dv, mask=m)
        cnt = cnt + csum[15]

    for t in range(RCH // 16):
        csrc_v[pl.ds(cnt + t * 16, 16)] = zer16
        cdst_v[pl.ds(cnt + t * 16, 16)] = zer16

    cvec[...] = zer16 + cnt
    pltpu.sync_copy(cvec, cnt_hbm.at[wid])

    obase = wid * CAP
    ncg = (cnt + (RCH - 1)) // RCH

    def g_issue(c, ba, bb, semg):
        off = c * RCH
        pltpu.async_copy(xws_hbm.at[csrc_v.at[pl.ds(off, RCH)]], ba, semg)
        pltpu.async_copy(xwd_hbm.at[cdst_v.at[pl.ds(off, RCH)]], bb, semg)

    def g_wait(ba, bb, semg):
        pltpu.make_async_copy(xws_hbm.at[pl.ds(0, RCH)], ba, semg).wait()
        pltpu.make_async_copy(xws_hbm.at[pl.ds(0, RCH)], bb, semg).wait()

    def o_issue(c, ba, bb, semo):
        off = c * RCH
        pltpu.async_copy(ba, pres_hbm.at[pl.ds(obase + off, RCH)], semo)
        pltpu.async_copy(bb, pred_hbm.at[pl.ds(obase + off, RCH)], semo)
        pltpu.async_copy(
            cdst_v.at[pl.ds(off, RCH)],
            dstc_hbm.at[(obase + off) // BLK, 0, pl.ds((obase + off) % BLK, RCH)],
            semo,
        )

    def o_wait(ba, bb, semo):
        pltpu.make_async_copy(ba, pres_hbm.at[pl.ds(obase, RCH)], semo).wait()
        pltpu.make_async_copy(bb, pred_hbm.at[pl.ds(obase, RCH)], semo).wait()
        pltpu.make_async_copy(
            cdst_v.at[pl.ds(0, RCH)],
            dstc_hbm.at[obase // BLK, 0, pl.ds(0, RCH)],
            semo,
        ).wait()

    @pl.when(ncg > 0)
    def _():
        g_issue(jnp.int32(0), bufa, bufb, sema)

    @pl.when(ncg > 1)
    def _():
        g_issue(jnp.int32(1), bufa2, bufb2, semb)

    def pbody(p, carry):
        c0 = 2 * p
        g_wait(bufa, bufb, sema)
        o_issue(c0, bufa, bufb, semoa)

        @pl.when(c0 + 1 < ncg)
        def _():
            g_wait(bufa2, bufb2, semb)
            o_issue(c0 + 1, bufa2, bufb2, semob)

        @pl.when(c0 + 2 < ncg)
        def _():
            o_wait(bufa, bufb, semoa)
            g_issue(c0 + 2, bufa, bufb, sema)

        @pl.when(c0 + 3 < ncg)
        def _():
            o_wait(bufa2, bufb2, semob)
            g_issue(c0 + 3, bufa2, bufb2, semb)

        return carry

    lax.fori_loop(0, (ncg + 1) // 2, pbody, 0)

    last_even = lax.rem(ncg - 1, 2) == 0

    @pl.when((ncg >= 1) & last_even)
    def _():
        o_wait(bufa, bufb, semoa)

    @pl.when(ncg >= 2)
    def _():
        o_wait(bufa2, bufb2, semob)

        @pl.when(jnp.logical_not(last_even))
        def _():
            o_wait(bufa, bufb, semoa)


def _stage2(edge_index, xws, xwd):
    mesh = plsc.VectorSubcoreMesh(core_axis_name="c", subcore_axis_name="s")
    f = functools.partial(
        pl.kernel,
        mesh=mesh,
        out_type=[
            jax.ShapeDtypeStruct((NW * CAP, H1), jnp.float32),
            jax.ShapeDtypeStruct((NW * CAP, H1), jnp.float32),
            jax.ShapeDtypeStruct((MAXB, 1, BLK), jnp.int32),
            jax.ShapeDtypeStruct((NW, 16), jnp.int32),
        ],
        scratch_types=[
            pltpu.VMEM((EPW,), jnp.int32),
            pltpu.VMEM((EPW,), jnp.int32),
            pltpu.VMEM((CAP,), jnp.int32),
            pltpu.VMEM((CAP,), jnp.int32),
            pltpu.VMEM((RCH, H1), jnp.float32),
            pltpu.VMEM((RCH, H1), jnp.float32),
            pltpu.VMEM((RCH, H1), jnp.float32),
            pltpu.VMEM((RCH, H1), jnp.float32),
            pltpu.VMEM((16,), jnp.int32),
            pltpu.SemaphoreType.DMA,
            pltpu.SemaphoreType.DMA,
            pltpu.SemaphoreType.DMA,
            pltpu.SemaphoreType.DMA,
        ],
        compiler_params=pltpu.CompilerParams(needs_layout_passes=False),
    )(_sc_body)
    return f(edge_index, xws, xwd)


def _critic_body(cnt_s, pres_hbm, pred_hbm, dstc_hbm,
                 x_ref, act_ref,
                 wm2_ref, bm2_ref, wm3_ref, bm3_ref,
                 wa1_ref, ba1_ref, wa2_ref, ba2_ref,
                 wu1_ref, bu1_ref, wu2_ref, bu2_ref, wu3_ref, bu3_ref,
                 wact_ref, bact_ref, wh1_ref, bh1_ref, wh2_ref, bh2_ref,
                 wq_ref, bq_ref,
                 out_ref,
                 sbuf, dbuf, dstv, aggr, rows_s, drow_s, vlim_s, sem):
    aggr[...] = jnp.zeros_like(aggr)

    def touter(t, idx):
        cnt = cnt_s[t, 0]
        nb = (cnt + (BLK - 1)) // BLK

        def binner(b, idx2):
            rows_s[idx2] = t * CAP + b * BLK
            drow_s[idx2] = t * NBW + b
            vlim_s[idx2] = cnt - b * BLK
            return idx2 + 1

        return lax.fori_loop(0, nb, binner, idx)

    total = lax.fori_loop(0, NW, touter, jnp.int32(0))

    def issue(k):
        s = lax.rem(k, 2)
        row0 = pl.multiple_of(rows_s[k], BLK)
        dr = drow_s[k]
        pltpu.make_async_copy(pres_hbm.at[pl.ds(row0, BLK)], sbuf.at[s], sem.at[s, 0]).start()
        pltpu.make_async_copy(pred_hbm.at[pl.ds(row0, BLK)], dbuf.at[s], sem.at[s, 1]).start()
        pltpu.make_async_copy(dstc_hbm.at[dr], dstv.at[s], sem.at[s, 2]).start()

    def wait(k):
        s = lax.rem(k, 2)
        pltpu.make_async_copy(pres_hbm.at[pl.ds(0, BLK)], sbuf.at[s], sem.at[s, 0]).wait()
        pltpu.make_async_copy(pred_hbm.at[pl.ds(0, BLK)], dbuf.at[s], sem.at[s, 1]).wait()
        pltpu.make_async_copy(dstc_hbm.at[0], dstv.at[s], sem.at[s, 2]).wait()

    @pl.when(total > 0)
    def _():
        issue(jnp.int32(0))

    def kbody(k, carry):
        @pl.when(k + 1 < total)
        def _():
            issue(k + 1)

        wait(k)
        s = lax.rem(k, 2)
        vlim = vlim_s[k]
        rows = lax.broadcasted_iota(jnp.int32, (BLK, 1), 0)
        m1 = jnp.where(rows < vlim, _relu(sbuf[s] + dbuf[s]), 0.0)
        m2 = _relu(jnp.dot(m1, wm2_ref[...], preferred_element_type=jnp.float32)
                   + bm2_ref[...])
        m3 = (jnp.dot(m2, wm3_ref[...], preferred_element_type=jnp.float32)
              + bm3_ref[...])
        cols = lax.broadcasted_iota(jnp.int32, (1, BLK), 1)
        dsel = jnp.where(cols < vlim, dstv[s], N_AGENTS)
        oh = (lax.broadcasted_iota(jnp.int32, (N_AGENTS, BLK), 0) == dsel
              ).astype(jnp.float32)
        m3h = m3.astype(jnp.bfloat16).astype(jnp.float32)
        m3l = m3 - m3h
        aggr[...] = (aggr[...]
                     + jnp.dot(oh, m3h, preferred_element_type=jnp.float32)
                     + jnp.dot(oh, m3l, preferred_element_type=jnp.float32))
        return carry

    lax.fori_loop(0, total, kbody, 0)

    ag = aggr[...]
    a = _relu(jnp.dot(ag, wa1_ref[...], preferred_element_type=jnp.float32) + ba1_ref[...])
    a = _relu(jnp.dot(a, wa2_ref[...], preferred_element_type=jnp.float32) + ba2_ref[...])
    h = _relu(jnp.dot(x_ref[...], wu1_ref[:D, :], preferred_element_type=jnp.float32)
              + jnp.dot(a, wu1_ref[D:, :], preferred_element_type=jnp.float32)
              + bu1_ref[...])
    h = _relu(jnp.dot(h, wu2_ref[...], preferred_element_type=jnp.float32) + bu2_ref[...])
    h = jnp.dot(h, wu3_ref[...], preferred_element_type=jnp.float32) + bu3_ref[...]
    ap = _relu(jnp.dot(act_ref[...], wact_ref[...], preferred_element_type=jnp.float32)
               + bact_ref[...])
    z = _relu(jnp.dot(h, wh1_ref[:D, :], preferred_element_type=jnp.float32)
              + jnp.dot(ap, wh1_ref[D:, :], preferred_element_type=jnp.float32)
              + bh1_ref[...])
    z = _relu(jnp.dot(z, wh2_ref[...], preferred_element_type=jnp.float32) + bh2_ref[...])
    zb = z.astype(jnp.bfloat16).astype(jnp.float32)
    wqb = wq_ref[...].astype(jnp.bfloat16).astype(jnp.float32)
    q = jnp.sum(zb * wqb, axis=1, keepdims=True) + bq_ref[...]
    out_ref[...] = q


def _stage3(counts, pres, pred, dstc2, x, actions, weights):
    in_specs = [pl.BlockSpec(memory_space=pltpu.SMEM),
                pl.BlockSpec(memory_space=pl.ANY),
                pl.BlockSpec(memory_space=pl.ANY),
                pl.BlockSpec(memory_space=pl.ANY),
                pl.BlockSpec(memory_space=pltpu.VMEM)]
    in_specs += [pl.BlockSpec(memory_space=pltpu.VMEM)] * (1 + len(weights))
    return pl.pallas_call(
        _critic_body,
        out_shape=jax.ShapeDtypeStruct((N_AGENTS, 1), jnp.float32),
        in_specs=in_specs,
        out_specs=pl.BlockSpec(memory_space=pltpu.VMEM),
        scratch_shapes=[
            pltpu.VMEM((2, BLK, H1), jnp.float32),
            pltpu.VMEM((2, BLK, H1), jnp.float32),
            pltpu.VMEM((2, 1, BLK), jnp.int32),
            pltpu.VMEM((N_AGENTS, D), jnp.float32),
            pltpu.SMEM((MAXB,), jnp.int32),
            pltpu.SMEM((MAXB,), jnp.int32),
            pltpu.SMEM((MAXB,), jnp.int32),
            pltpu.SemaphoreType.DMA((2, 3)),
        ],
    )(counts, pres, pred, dstc2, x, actions, *weights)


def kernel(x, edge_index, actions,
           Wm1, bm1, Wm2, bm2, Wm3, bm3,
           Wa1, ba1, Wa2, ba2,
           Wu1, bu1, Wu2, bu2, Wu3, bu3,
           Wact, bact, Wh1, bh1, Wh2, bh2, Wq, bq):
    xws, xwd = _stage1(x, Wm1, bm1.reshape(1, -1))
    pres, pred, dstc2, counts = _stage2(edge_index.reshape(-1), xws, xwd)
    weights = (Wm2, bm2.reshape(1, -1), Wm3, bm3.reshape(1, -1),
               Wa1, ba1.reshape(1, -1), Wa2, ba2.reshape(1, -1),
               Wu1, bu1.reshape(1, -1), Wu2, bu2.reshape(1, -1),
               Wu3, bu3.reshape(1, -1),
               Wact, bact.reshape(1, -1),
               Wh1, bh1.reshape(1, -1), Wh2, bh2.reshape(1, -1),
               Wq.reshape(1, -1), bq.reshape(1, 1))
    out = _stage3(counts, pres, pred, dstc2, x[:N_AGENTS], actions, weights)
    return out.reshape(N_AGENTS)
